# Initial kernel scaffold; baseline (speedup 1.0000x reference)
import jax
import jax.numpy as jnp
from jax.experimental import pallas as pl


def kernel(x, edge_index, edge_weight, Wz, bz, Wr, br, Wh, bh, Lz_w, Lz_b, Lr_w, Lr_b, Lh_w, Lh_b, out_w, out_b):
    raise NotImplementedError("write your pallas kernel here")


# speedup vs baseline: 55.3239x; 55.3239x over previous
import dataclasses
import functools

import jax
import jax.numpy as jnp
from jax import lax
from jax.experimental import pallas as pl
from jax.experimental.pallas import tpu as pltpu
from jax.experimental.pallas import tpu_sc as plsc

N = 10000
NP = 10240
E = 160000
F_IN = 256
F_OUT = 32
FC = 2 * F_OUT
FP = 2 * FC
B = 8
NPAIR = B // 2

NC = 2
NS = 16
CH = 128
E_PAD = 163840
NCH_AGG = E_PAD // NS // CH
EPT_DEG = E_PAD // (NC * NS)
RPT = NP // NS
PPC = NPAIR // NC

_MM_BLK = 2048
_NB = NP // _MM_BLK

_mesh = plsc.VectorSubcoreMesh(core_axis_name="c", subcore_axis_name="s")

_sc_params = pltpu.CompilerParams()
if "needs_layout_passes" in pltpu.CompilerParams.__dataclass_fields__:
    _sc_params = dataclasses.replace(_sc_params, needs_layout_passes=False)


def _fold_body(wz, lz, wh, lh, bz, lzb, bh, lhb, wf, cf):
    lzt = lz[0:F_OUT, :]
    lht = lh[0:F_OUT, :]
    wf[...] = jnp.concatenate(
        [jnp.dot(wz[...], lzt, preferred_element_type=jnp.float32),
         jnp.dot(wh[...], lht, preferred_element_type=jnp.float32)], axis=1)
    cz = jnp.dot(bz[...], lzt, preferred_element_type=jnp.float32) + lzb[...]
    ch = jnp.dot(bh[...], lht, preferred_element_type=jnp.float32) + lhb[...]
    cf[...] = jnp.concatenate([cz, ch], axis=1)


_fold = pl.pallas_call(
    _fold_body,
    out_shape=[jax.ShapeDtypeStruct((F_IN, FC), jnp.float32),
               jax.ShapeDtypeStruct((1, FC), jnp.float32)],
)


@functools.partial(
    pl.kernel,
    out_type=jax.ShapeDtypeStruct((NC, NP), jnp.float32),
    mesh=_mesh,
    compiler_params=_sc_params,
    scratch_types=[
        pltpu.VMEM_SHARED((NS, NP), jnp.float32),
        pltpu.VMEM((NP,), jnp.float32),
        pltpu.VMEM((EPT_DEG,), jnp.int32),
        pltpu.VMEM((EPT_DEG,), jnp.float32),
        pltpu.VMEM((NS, RPT), jnp.float32),
        pltpu.VMEM((RPT,), jnp.float32),
    ])
def _deg_kernel(col_hbm, ew_hbm, deg_hbm, shp, degloc, colv, ewv, tbuf, sumb):
    c = lax.axis_index("c")
    s = lax.axis_index("s")
    pltpu.sync_copy(col_hbm.at[c, s], colv)
    pltpu.sync_copy(ew_hbm.at[c, s], ewv)

    zero16 = jnp.zeros((16,), jnp.float32)

    @pl.loop(0, NP // 16)
    def _z(i):
        degloc[pl.ds(i * 16, 16)] = zero16

    @pl.loop(0, EPT_DEG // 16)
    def _h(k):
        idx = colv[pl.ds(k * 16, 16)]
        w = ewv[pl.ds(k * 16, 16)]
        plsc.addupdate_scatter(degloc, [idx], w)

    pltpu.sync_copy(degloc, shp.at[s])
    plsc.subcore_barrier()
    for k in range(NS):
        pltpu.sync_copy(shp.at[k, pl.ds(s * RPT, RPT)], tbuf.at[k])

    @pl.loop(0, RPT // 16)
    def _r(j):
        acc = tbuf[0, pl.ds(j * 16, 16)]
        for k in range(1, NS):
            acc = acc + tbuf[k, pl.ds(j * 16, 16)]
        sumb[pl.ds(j * 16, 16)] = acc

    pltpu.sync_copy(sumb, deg_hbm.at[c, pl.ds(s * RPT, RPT)])


def _mm_body(xa_ref, xb_ref, wf_ref, deg_ref, y_ref):
    degs = deg_ref[0] + deg_ref[1]
    dinv = lax.rsqrt(1.0 + degs)[:, None]
    ya = jnp.dot(xa_ref[...], wf_ref[...], preferred_element_type=jnp.float32)
    yb = jnp.dot(xb_ref[...], wf_ref[...], preferred_element_type=jnp.float32)
    y_ref[...] = jnp.concatenate([ya * dinv, yb * dinv], axis=1)[None]


def _matmul(xr, wf, degp):
    return pl.pallas_call(
        _mm_body,
        grid=(NPAIR, _NB),
        in_specs=[
            pl.BlockSpec((_MM_BLK, F_IN), lambda p, i: (2 * p * _NB + i, 0)),
            pl.BlockSpec((_MM_BLK, F_IN),
                         lambda p, i: ((2 * p + 1) * _NB + i, 0)),
            pl.BlockSpec((F_IN, FC), lambda p, i: (0, 0)),
            pl.BlockSpec((NC, _MM_BLK), lambda p, i: (0, i)),
        ],
        out_specs=pl.BlockSpec((1, _MM_BLK, FP), lambda p, i: (p, i, 0)),
        out_shape=jax.ShapeDtypeStruct((NPAIR, NP, FP), jnp.float32),
    )(xr, xr, wf, degp)


@functools.partial(
    pl.kernel,
    out_type=jax.ShapeDtypeStruct((NPAIR * NP, FP), jnp.float32),
    mesh=_mesh,
    compiler_params=_sc_params,
    scratch_types=[
        pltpu.VMEM_SHARED((NP, FP), jnp.float32),
        pltpu.VMEM((NCH_AGG, CH), jnp.int32),
        pltpu.VMEM((NCH_AGG, CH), jnp.int32),
        pltpu.VMEM((NCH_AGG, CH), jnp.float32),
        pltpu.VMEM((CH,), jnp.int32),
        pltpu.VMEM((CH,), jnp.int32),
        pltpu.VMEM((CH,), jnp.float32),
        pltpu.VMEM((CH, FP), jnp.float32),
    ])
def _agg_kernel(y_hbm, row_hbm, col_hbm, ew_hbm, s_hbm,
                acc, rowv, colv, ewv, ridx, cidx, ew1, gbuf):
    c = lax.axis_index("c")
    s = lax.axis_index("s")
    pltpu.sync_copy(row_hbm.at[s], rowv)
    pltpu.sync_copy(col_hbm.at[s], colv)
    pltpu.sync_copy(ew_hbm.at[s], ewv)
    rbase = s * RPT
    for q in range(PPC):
        p = c * PPC + q
        pbase = p * NP
        pltpu.sync_copy(y_hbm.at[pl.ds(pbase + rbase, RPT)],
                        acc.at[pl.ds(rbase, RPT)])
        plsc.subcore_barrier()

        @pl.loop(0, NCH_AGG)
        def _chunk(i):
            for j in range(CH // 16):
                sl = pl.ds(j * 16, 16)
                ridx[sl] = rowv[i, sl] + pbase
                cidx[sl] = colv[i, sl]
                ew1[sl] = ewv[i, sl]
            pltpu.sync_copy(y_hbm.at[ridx], gbuf)

            @pl.loop(0, CH)
            def _edge(e):
                w = plsc.load_gather(ew1, [jnp.full((16,), e, jnp.int32)])
                for f in range(FP // 16):
                    fsl = (e, pl.ds(f * 16, 16))
                    gbuf[fsl] = gbuf[fsl] * w

            pltpu.sync_copy(gbuf, acc.at[cidx], add=True)

        plsc.subcore_barrier()
        pltpu.sync_copy(acc.at[pl.ds(rbase, RPT)],
                        s_hbm.at[pl.ds(pbase + rbase, RPT)])
        plsc.subcore_barrier()


def _post_body(acc_ref, deg_ref, cf_ref, ow_ref, ob_ref, o_ref):
    degs = deg_ref[0] + deg_ref[1]
    dinv = lax.rsqrt(1.0 + degs)[:, None]
    v = acc_ref[...] * dinv
    outs = []
    for q in range(2):
        z = jax.nn.sigmoid(v[:, q * FC:q * FC + F_OUT]
                           + cf_ref[0, 0:F_OUT])
        ht = jnp.tanh(v[:, q * FC + F_OUT:(q + 1) * FC]
                      + cf_ref[0, F_OUT:FC])
        h = (1.0 - z) * ht
        outs.append(jnp.sum(h * ow_ref[0, :], axis=1, keepdims=True))
    o_ref[...] = jax.nn.sigmoid(jnp.concatenate(outs, axis=1) + ob_ref[...])


def _post(accr, degp, cf, owr, ob):
    return pl.pallas_call(
        _post_body,
        grid=(NPAIR * _NB,),
        in_specs=[
            pl.BlockSpec((_MM_BLK, FP), lambda i: (i, 0)),
            pl.BlockSpec((NC, _MM_BLK), lambda i: (0, i % _NB)),
            pl.BlockSpec((1, FC), lambda i: (0, 0)),
            pl.BlockSpec((1, F_OUT), lambda i: (0, 0)),
            pl.BlockSpec((1, 1), lambda i: (0, 0)),
        ],
        out_specs=pl.BlockSpec((_MM_BLK, 2), lambda i: (i, 0)),
        out_shape=jax.ShapeDtypeStruct((NPAIR * NP, 2), jnp.float32),
    )(accr, degp, cf, owr, ob)


def kernel(x, edge_index, edge_weight, Wz, bz, Wr, br, Wh, bh,
           Lz_w, Lz_b, Lr_w, Lr_b, Lh_w, Lh_b, out_w, out_b):
    ei = edge_index.astype(jnp.int32)
    pad = E_PAD - E
    row = jnp.concatenate([ei[0], jnp.zeros((pad,), jnp.int32)])
    col = jnp.concatenate([ei[1], jnp.zeros((pad,), jnp.int32)])
    ew = jnp.concatenate([edge_weight.astype(jnp.float32),
                          jnp.zeros((pad,), jnp.float32)])

    wf, cf = _fold(Wz, Lz_w, Wh, Lh_w,
                   bz.reshape(1, F_OUT), Lz_b.reshape(1, F_OUT),
                   bh.reshape(1, F_OUT), Lh_b.reshape(1, F_OUT))
    degp = _deg_kernel(col.reshape(NC, NS, EPT_DEG),
                       ew.reshape(NC, NS, EPT_DEG))
    xp = jnp.concatenate(
        [x, jnp.zeros((B, NP - N, F_IN), jnp.float32)], axis=1)
    y2 = _matmul(xp.reshape(B * NP, F_IN), wf, degp)
    accs = _agg_kernel(y2.reshape(NPAIR * NP, FP),
                       row.reshape(NS, NCH_AGG, CH),
                       col.reshape(NS, NCH_AGG, CH),
                       ew.reshape(NS, NCH_AGG, CH))
    o = _post(accs, degp, cf, out_w.reshape(1, F_OUT), out_b.reshape(1, 1))
    return o.reshape(NPAIR, NP, 2).transpose(0, 2, 1).reshape(B, NP, 1)[:, :N]


# speedup vs baseline: 60.9821x; 1.1023x over previous
import dataclasses
import functools

import jax
import jax.numpy as jnp
from jax import lax
from jax.experimental import pallas as pl
from jax.experimental.pallas import tpu as pltpu
from jax.experimental.pallas import tpu_sc as plsc

N = 10000
NP = 10240
E = 160000
F_IN = 256
F_OUT = 32
FC = 2 * F_OUT
FP = 2 * FC
B = 8
NPAIR = B // 2

NC = 2
NS = 16
CH = 128
E_PAD = 163840
NCH_AGG = E_PAD // NS // CH
EPT_DEG = E_PAD // (NC * NS)
RPT = NP // NS
PPC = NPAIR // NC

_MM_BLK = 2048
_NB = NP // _MM_BLK

_mesh = plsc.VectorSubcoreMesh(core_axis_name="c", subcore_axis_name="s")

_sc_params = pltpu.CompilerParams()
if "needs_layout_passes" in pltpu.CompilerParams.__dataclass_fields__:
    _sc_params = dataclasses.replace(_sc_params, needs_layout_passes=False)


def _fold_body(wz, lz, wh, lh, bz, lzb, bh, lhb, wf, cf):
    lzt = lz[0:F_OUT, :]
    lht = lh[0:F_OUT, :]
    wf[...] = jnp.concatenate(
        [jnp.dot(wz[...], lzt, preferred_element_type=jnp.float32),
         jnp.dot(wh[...], lht, preferred_element_type=jnp.float32)], axis=1)
    cz = jnp.dot(bz[...], lzt, preferred_element_type=jnp.float32) + lzb[...]
    ch = jnp.dot(bh[...], lht, preferred_element_type=jnp.float32) + lhb[...]
    cf[...] = jnp.concatenate([cz, ch], axis=1)


_fold = pl.pallas_call(
    _fold_body,
    out_shape=[jax.ShapeDtypeStruct((F_IN, FC), jnp.float32),
               jax.ShapeDtypeStruct((1, FC), jnp.float32)],
)


@functools.partial(
    pl.kernel,
    out_type=jax.ShapeDtypeStruct((NC, NP), jnp.float32),
    mesh=_mesh,
    compiler_params=_sc_params,
    scratch_types=[
        pltpu.VMEM_SHARED((NS, NP), jnp.float32),
        pltpu.VMEM((NP,), jnp.float32),
        pltpu.VMEM((EPT_DEG,), jnp.int32),
        pltpu.VMEM((EPT_DEG,), jnp.float32),
        pltpu.VMEM((NS, RPT), jnp.float32),
        pltpu.VMEM((RPT,), jnp.float32),
    ])
def _deg_kernel(col_hbm, ew_hbm, deg_hbm, shp, degloc, colv, ewv, tbuf, sumb):
    c = lax.axis_index("c")
    s = lax.axis_index("s")
    pltpu.sync_copy(col_hbm.at[c, s], colv)
    pltpu.sync_copy(ew_hbm.at[c, s], ewv)

    zero16 = jnp.zeros((16,), jnp.float32)

    @pl.loop(0, NP // 16)
    def _z(i):
        degloc[pl.ds(i * 16, 16)] = zero16

    @pl.loop(0, EPT_DEG // 16)
    def _h(k):
        idx = colv[pl.ds(k * 16, 16)]
        w = ewv[pl.ds(k * 16, 16)]
        plsc.addupdate_scatter(degloc, [idx], w)

    pltpu.sync_copy(degloc, shp.at[s])
    plsc.subcore_barrier()
    for k in range(NS):
        pltpu.sync_copy(shp.at[k, pl.ds(s * RPT, RPT)], tbuf.at[k])

    @pl.loop(0, RPT // 16)
    def _r(j):
        acc = tbuf[0, pl.ds(j * 16, 16)]
        for k in range(1, NS):
            acc = acc + tbuf[k, pl.ds(j * 16, 16)]
        sumb[pl.ds(j * 16, 16)] = acc

    pltpu.sync_copy(sumb, deg_hbm.at[c, pl.ds(s * RPT, RPT)])


def _mm_body(xa_ref, xb_ref, wf_ref, deg_ref, y_ref):
    degs = deg_ref[0] + deg_ref[1]
    dinv = lax.rsqrt(1.0 + degs)[:, None]
    ya = jnp.dot(xa_ref[...], wf_ref[...], preferred_element_type=jnp.float32)
    yb = jnp.dot(xb_ref[...], wf_ref[...], preferred_element_type=jnp.float32)
    y_ref[...] = jnp.concatenate([ya * dinv, yb * dinv], axis=1)[None]


def _matmul(xr, wf, degp):
    return pl.pallas_call(
        _mm_body,
        grid=(NPAIR, _NB),
        in_specs=[
            pl.BlockSpec((_MM_BLK, F_IN), lambda p, i: (2 * p * _NB + i, 0)),
            pl.BlockSpec((_MM_BLK, F_IN),
                         lambda p, i: ((2 * p + 1) * _NB + i, 0)),
            pl.BlockSpec((F_IN, FC), lambda p, i: (0, 0)),
            pl.BlockSpec((NC, _MM_BLK), lambda p, i: (0, i)),
        ],
        out_specs=pl.BlockSpec((1, _MM_BLK, FP), lambda p, i: (p, i, 0)),
        out_shape=jax.ShapeDtypeStruct((NPAIR, NP, FP), jnp.float32),
    )(xr, xr, wf, degp)


@functools.partial(
    pl.kernel,
    out_type=jax.ShapeDtypeStruct((NPAIR * NP, FP), jnp.float32),
    mesh=_mesh,
    compiler_params=_sc_params,
    scratch_types=[
        pltpu.VMEM_SHARED((NP, FP), jnp.float32),
        pltpu.VMEM((NCH_AGG, CH), jnp.int32),
        pltpu.VMEM((NCH_AGG, CH), jnp.int32),
        pltpu.VMEM((NCH_AGG, CH), jnp.float32),
        pltpu.VMEM((CH,), jnp.int32),
        pltpu.VMEM((CH,), jnp.int32),
        pltpu.VMEM((CH,), jnp.float32),
        pltpu.VMEM((CH, FP), jnp.float32),
    ])
def _agg_kernel(y_hbm, row_hbm, col_hbm, ew_hbm, s_hbm,
                acc, rowv, colv, ewv, ridx, cidx, ew1, gbuf):
    c = lax.axis_index("c")
    s = lax.axis_index("s")
    pltpu.sync_copy(row_hbm.at[s], rowv)
    pltpu.sync_copy(col_hbm.at[s], colv)
    pltpu.sync_copy(ew_hbm.at[s], ewv)
    rbase = s * RPT
    for q in range(PPC):
        p = c * PPC + q
        pbase = p * NP
        pltpu.sync_copy(y_hbm.at[pl.ds(pbase + rbase, RPT)],
                        acc.at[pl.ds(rbase, RPT)])
        plsc.subcore_barrier()

        @pl.loop(0, NCH_AGG)
        def _chunk(i):
            for j in range(CH // 16):
                sl = pl.ds(j * 16, 16)
                ridx[sl] = rowv[i, sl] + pbase
                cidx[sl] = colv[i, sl]
                ew1[sl] = ewv[i, sl]
            pltpu.sync_copy(y_hbm.at[ridx], gbuf)

            @pl.loop(0, CH // 16)
            def _grp(j):
                w16 = ew1[pl.ds(j * 16, 16)]
                base = j * 16
                for l in range(16):
                    w = jnp.broadcast_to(w16[l], (16,))
                    for f in range(FP // 16):
                        fsl = (base + l, pl.ds(f * 16, 16))
                        gbuf[fsl] = gbuf[fsl] * w

            pltpu.sync_copy(gbuf, acc.at[cidx], add=True)

        plsc.subcore_barrier()
        pltpu.sync_copy(acc.at[pl.ds(rbase, RPT)],
                        s_hbm.at[pl.ds(pbase + rbase, RPT)])
        plsc.subcore_barrier()


def _post_body(acc_ref, deg_ref, cf_ref, ow_ref, ob_ref, o_ref):
    degs = deg_ref[0] + deg_ref[1]
    dinv = lax.rsqrt(1.0 + degs)[:, None]
    v = acc_ref[...] * dinv
    outs = []
    for q in range(2):
        z = jax.nn.sigmoid(v[:, q * FC:q * FC + F_OUT]
                           + cf_ref[0, 0:F_OUT])
        ht = jnp.tanh(v[:, q * FC + F_OUT:(q + 1) * FC]
                      + cf_ref[0, F_OUT:FC])
        h = (1.0 - z) * ht
        outs.append(jnp.sum(h * ow_ref[0, :], axis=1, keepdims=True))
    o_ref[...] = jax.nn.sigmoid(jnp.concatenate(outs, axis=1) + ob_ref[...])


def _post(accr, degp, cf, owr, ob):
    return pl.pallas_call(
        _post_body,
        grid=(NPAIR * _NB,),
        in_specs=[
            pl.BlockSpec((_MM_BLK, FP), lambda i: (i, 0)),
            pl.BlockSpec((NC, _MM_BLK), lambda i: (0, i % _NB)),
            pl.BlockSpec((1, FC), lambda i: (0, 0)),
            pl.BlockSpec((1, F_OUT), lambda i: (0, 0)),
            pl.BlockSpec((1, 1), lambda i: (0, 0)),
        ],
        out_specs=pl.BlockSpec((_MM_BLK, 2), lambda i: (i, 0)),
        out_shape=jax.ShapeDtypeStruct((NPAIR * NP, 2), jnp.float32),
    )(accr, degp, cf, owr, ob)


def kernel(x, edge_index, edge_weight, Wz, bz, Wr, br, Wh, bh,
           Lz_w, Lz_b, Lr_w, Lr_b, Lh_w, Lh_b, out_w, out_b):
    ei = edge_index.astype(jnp.int32)
    pad = E_PAD - E
    row = jnp.concatenate([ei[0], jnp.zeros((pad,), jnp.int32)])
    col = jnp.concatenate([ei[1], jnp.zeros((pad,), jnp.int32)])
    ew = jnp.concatenate([edge_weight.astype(jnp.float32),
                          jnp.zeros((pad,), jnp.float32)])

    wf, cf = _fold(Wz, Lz_w, Wh, Lh_w,
                   bz.reshape(1, F_OUT), Lz_b.reshape(1, F_OUT),
                   bh.reshape(1, F_OUT), Lh_b.reshape(1, F_OUT))
    degp = _deg_kernel(col.reshape(NC, NS, EPT_DEG),
                       ew.reshape(NC, NS, EPT_DEG))
    xp = jnp.concatenate(
        [x, jnp.zeros((B, NP - N, F_IN), jnp.float32)], axis=1)
    y2 = _matmul(xp.reshape(B * NP, F_IN), wf, degp)
    accs = _agg_kernel(y2.reshape(NPAIR * NP, FP),
                       row.reshape(NS, NCH_AGG, CH),
                       col.reshape(NS, NCH_AGG, CH),
                       ew.reshape(NS, NCH_AGG, CH))
    o = _post(accs, degp, cf, out_w.reshape(1, F_OUT), out_b.reshape(1, 1))
    return o.reshape(NPAIR, NP, 2).transpose(0, 2, 1).reshape(B, NP, 1)[:, :N]


# speedup vs baseline: 68.3916x; 1.1215x over previous
import dataclasses
import functools

import jax
import jax.numpy as jnp
from jax import lax
from jax.experimental import pallas as pl
from jax.experimental.pallas import tpu as pltpu
from jax.experimental.pallas import tpu_sc as plsc

N = 10000
NP = 10240
E = 160000
F_IN = 256
F_OUT = 32
FC = 2 * F_OUT
FP = 2 * FC
B = 8
NPAIR = B // 2

NC = 2
NS = 16
CH = 128
E_PAD = 163840
NCH_AGG = E_PAD // NS // CH
EPT_DEG = E_PAD // (NC * NS)
RPT = NP // NS
PPC = NPAIR // NC

_MM_BLK = 2048
_NB = NP // _MM_BLK

_mesh = plsc.VectorSubcoreMesh(core_axis_name="c", subcore_axis_name="s")

_sc_params = pltpu.CompilerParams()
if "needs_layout_passes" in pltpu.CompilerParams.__dataclass_fields__:
    _sc_params = dataclasses.replace(_sc_params, needs_layout_passes=False)


def _fold_body(wz, lz, wh, lh, bz, lzb, bh, lhb, wf, cf):
    lzt = lz[0:F_OUT, :]
    lht = lh[0:F_OUT, :]
    wf[...] = jnp.concatenate(
        [jnp.dot(wz[...], lzt, preferred_element_type=jnp.float32),
         jnp.dot(wh[...], lht, preferred_element_type=jnp.float32)], axis=1)
    cz = jnp.dot(bz[...], lzt, preferred_element_type=jnp.float32) + lzb[...]
    ch = jnp.dot(bh[...], lht, preferred_element_type=jnp.float32) + lhb[...]
    cf[...] = jnp.concatenate([cz, ch], axis=1)


_fold = pl.pallas_call(
    _fold_body,
    out_shape=[jax.ShapeDtypeStruct((F_IN, FC), jnp.float32),
               jax.ShapeDtypeStruct((1, FC), jnp.float32)],
)


@functools.partial(
    pl.kernel,
    out_type=jax.ShapeDtypeStruct((NC, NP), jnp.float32),
    mesh=_mesh,
    compiler_params=_sc_params,
    scratch_types=[
        pltpu.VMEM_SHARED((NS, NP), jnp.float32),
        pltpu.VMEM((NP,), jnp.float32),
        pltpu.VMEM((EPT_DEG,), jnp.int32),
        pltpu.VMEM((EPT_DEG,), jnp.float32),
        pltpu.VMEM((NS, RPT), jnp.float32),
        pltpu.VMEM((RPT,), jnp.float32),
    ])
def _deg_kernel(col_hbm, ew_hbm, deg_hbm, shp, degloc, colv, ewv, tbuf, sumb):
    c = lax.axis_index("c")
    s = lax.axis_index("s")
    pltpu.sync_copy(col_hbm.at[c, s], colv)
    pltpu.sync_copy(ew_hbm.at[c, s], ewv)

    zero16 = jnp.zeros((16,), jnp.float32)

    @pl.loop(0, NP // 16)
    def _z(i):
        degloc[pl.ds(i * 16, 16)] = zero16

    @pl.loop(0, EPT_DEG // 16)
    def _h(k):
        idx = colv[pl.ds(k * 16, 16)]
        w = ewv[pl.ds(k * 16, 16)]
        plsc.addupdate_scatter(degloc, [idx], w)

    pltpu.sync_copy(degloc, shp.at[s])
    plsc.subcore_barrier()
    for k in range(NS):
        pltpu.sync_copy(shp.at[k, pl.ds(s * RPT, RPT)], tbuf.at[k])

    @pl.loop(0, RPT // 16)
    def _r(j):
        acc = tbuf[0, pl.ds(j * 16, 16)]
        for k in range(1, NS):
            acc = acc + tbuf[k, pl.ds(j * 16, 16)]
        sumb[pl.ds(j * 16, 16)] = acc

    pltpu.sync_copy(sumb, deg_hbm.at[c, pl.ds(s * RPT, RPT)])


def _mm_body(xa_ref, xb_ref, wf_ref, deg_ref, y_ref):
    degs = deg_ref[0] + deg_ref[1]
    dinv = lax.rsqrt(1.0 + degs)[:, None]
    ya = jnp.dot(xa_ref[...], wf_ref[...], preferred_element_type=jnp.float32)
    yb = jnp.dot(xb_ref[...], wf_ref[...], preferred_element_type=jnp.float32)
    y_ref[...] = jnp.concatenate([ya * dinv, yb * dinv], axis=1)[None]


def _matmul(xr, wf, degp):
    return pl.pallas_call(
        _mm_body,
        grid=(NPAIR, _NB),
        in_specs=[
            pl.BlockSpec((_MM_BLK, F_IN), lambda p, i: (2 * p * _NB + i, 0)),
            pl.BlockSpec((_MM_BLK, F_IN),
                         lambda p, i: ((2 * p + 1) * _NB + i, 0)),
            pl.BlockSpec((F_IN, FC), lambda p, i: (0, 0)),
            pl.BlockSpec((NC, _MM_BLK), lambda p, i: (0, i)),
        ],
        out_specs=pl.BlockSpec((1, _MM_BLK, FP), lambda p, i: (p, i, 0)),
        out_shape=jax.ShapeDtypeStruct((NPAIR, NP, FP), jnp.float32),
    )(xr, xr, wf, degp)


@functools.partial(
    pl.kernel,
    out_type=jax.ShapeDtypeStruct((NPAIR * NP, FP), jnp.float32),
    mesh=_mesh,
    compiler_params=_sc_params,
    scratch_types=[
        pltpu.VMEM_SHARED((NP, FP), jnp.float32),
        pltpu.VMEM((NCH_AGG, CH), jnp.float32),
        pltpu.VMEM((CH,), jnp.int32),
        pltpu.VMEM((CH,), jnp.int32),
        pltpu.VMEM((CH,), jnp.int32),
        pltpu.VMEM((CH,), jnp.int32),
        pltpu.VMEM((CH, FP), jnp.float32),
        pltpu.VMEM((CH, FP), jnp.float32),
        pltpu.SemaphoreType.DMA,
        pltpu.SemaphoreType.DMA,
        pltpu.SemaphoreType.DMA,
        pltpu.SemaphoreType.DMA,
    ])
def _agg_kernel(y_hbm, row_hbm, col_hbm, ew_hbm, s_hbm,
                acc, ewv, ridx0, ridx1, cidx0, cidx1,
                gbuf0, gbuf1, gsem0, gsem1, ssem0, ssem1):
    c = lax.axis_index("c")
    s = lax.axis_index("s")
    pltpu.sync_copy(ew_hbm.at[s], ewv)
    rbase = s * RPT

    def _scale(gbuf, i):
        @pl.loop(0, CH // 16)
        def _grp(j):
            w16 = ewv[i, pl.ds(j * 16, 16)]
            base = j * 16
            for l in range(16):
                w = jnp.broadcast_to(w16[l], (16,))
                for f in range(FP // 16):
                    fsl = (base + l, pl.ds(f * 16, 16))
                    gbuf[fsl] = gbuf[fsl] * w

    for q in range(PPC):
        p = c * PPC + q
        pbase = p * NP
        pltpu.sync_copy(y_hbm.at[pl.ds(pbase + rbase, RPT)],
                        acc.at[pl.ds(rbase, RPT)])
        plsc.subcore_barrier()

        bufs = ((ridx0, cidx0, gbuf0, gsem0, ssem0),
                (ridx1, cidx1, gbuf1, gsem1, ssem1))

        @pl.loop(0, NCH_AGG // 2)
        def _chunk(k):
            for b, (ridx, cidx, gbuf, gsem, ssem) in enumerate(bufs):
                i = 2 * k + b

                @pl.when(k > 0)
                def _drain():
                    pltpu.make_async_copy(gbuf, acc.at[cidx], ssem).wait()

                pltpu.sync_copy(row_hbm.at[s, i], ridx)
                pltpu.sync_copy(col_hbm.at[s, i], cidx)
                for j in range(CH // 16):
                    sl = pl.ds(j * 16, 16)
                    ridx[sl] = ridx[sl] + pbase
                pltpu.async_copy(y_hbm.at[ridx], gbuf, gsem)

            for b, (ridx, cidx, gbuf, gsem, ssem) in enumerate(bufs):
                i = 2 * k + b
                pltpu.make_async_copy(y_hbm.at[ridx], gbuf, gsem).wait()
                _scale(gbuf, i)
                pltpu.async_copy(gbuf, acc.at[cidx], ssem, add=True)

        for ridx, cidx, gbuf, gsem, ssem in bufs:
            pltpu.make_async_copy(gbuf, acc.at[cidx], ssem).wait()
        plsc.subcore_barrier()
        pltpu.sync_copy(acc.at[pl.ds(rbase, RPT)],
                        s_hbm.at[pl.ds(pbase + rbase, RPT)])
        plsc.subcore_barrier()


def _post_body(acc_ref, deg_ref, cf_ref, ow_ref, ob_ref, o_ref):
    degs = deg_ref[0] + deg_ref[1]
    dinv = lax.rsqrt(1.0 + degs)[:, None]
    v = acc_ref[...] * dinv
    outs = []
    for q in range(2):
        z = jax.nn.sigmoid(v[:, q * FC:q * FC + F_OUT]
                           + cf_ref[0, 0:F_OUT])
        ht = jnp.tanh(v[:, q * FC + F_OUT:(q + 1) * FC]
                      + cf_ref[0, F_OUT:FC])
        h = (1.0 - z) * ht
        outs.append(jnp.sum(h * ow_ref[0, :], axis=1, keepdims=True))
    o_ref[...] = jax.nn.sigmoid(jnp.concatenate(outs, axis=1) + ob_ref[...])


def _post(accr, degp, cf, owr, ob):
    return pl.pallas_call(
        _post_body,
        grid=(NPAIR * _NB,),
        in_specs=[
            pl.BlockSpec((_MM_BLK, FP), lambda i: (i, 0)),
            pl.BlockSpec((NC, _MM_BLK), lambda i: (0, i % _NB)),
            pl.BlockSpec((1, FC), lambda i: (0, 0)),
            pl.BlockSpec((1, F_OUT), lambda i: (0, 0)),
            pl.BlockSpec((1, 1), lambda i: (0, 0)),
        ],
        out_specs=pl.BlockSpec((_MM_BLK, 2), lambda i: (i, 0)),
        out_shape=jax.ShapeDtypeStruct((NPAIR * NP, 2), jnp.float32),
    )(accr, degp, cf, owr, ob)


def kernel(x, edge_index, edge_weight, Wz, bz, Wr, br, Wh, bh,
           Lz_w, Lz_b, Lr_w, Lr_b, Lh_w, Lh_b, out_w, out_b):
    ei = edge_index.astype(jnp.int32)
    pad = E_PAD - E
    row = jnp.concatenate([ei[0], jnp.zeros((pad,), jnp.int32)])
    col = jnp.concatenate([ei[1], jnp.zeros((pad,), jnp.int32)])
    ew = jnp.concatenate([edge_weight.astype(jnp.float32),
                          jnp.zeros((pad,), jnp.float32)])

    wf, cf = _fold(Wz, Lz_w, Wh, Lh_w,
                   bz.reshape(1, F_OUT), Lz_b.reshape(1, F_OUT),
                   bh.reshape(1, F_OUT), Lh_b.reshape(1, F_OUT))
    degp = _deg_kernel(col.reshape(NC, NS, EPT_DEG),
                       ew.reshape(NC, NS, EPT_DEG))
    xp = jnp.concatenate(
        [x, jnp.zeros((B, NP - N, F_IN), jnp.float32)], axis=1)
    y2 = _matmul(xp.reshape(B * NP, F_IN), wf, degp)
    accs = _agg_kernel(y2.reshape(NPAIR * NP, FP),
                       row.reshape(NS, NCH_AGG, CH),
                       col.reshape(NS, NCH_AGG, CH),
                       ew.reshape(NS, NCH_AGG, CH))
    o = _post(accs, degp, cf, out_w.reshape(1, F_OUT), out_b.reshape(1, 1))
    return o.reshape(NPAIR, NP, 2).transpose(0, 2, 1).reshape(B, NP, 1)[:, :N]


# speedup vs baseline: 72.5198x; 1.0604x over previous
import dataclasses
import functools

import jax
import jax.numpy as jnp
from jax import lax
from jax.experimental import pallas as pl
from jax.experimental.pallas import tpu as pltpu
from jax.experimental.pallas import tpu_sc as plsc

N = 10000
NP = 10240
E = 160000
F_IN = 256
F_OUT = 32
FC = 2 * F_OUT
FP = 2 * FC
B = 8
NPAIR = B // 2

NC = 2
NS = 16
CH = 128
E_PAD = 163840
NCH_AGG = E_PAD // NS // CH
EPT_DEG = E_PAD // (NC * NS)
RPT = NP // NS
PPC = NPAIR // NC

_MM_BLK = 2048
_NB = NP // _MM_BLK

_mesh = plsc.VectorSubcoreMesh(core_axis_name="c", subcore_axis_name="s")

_sc_params = pltpu.CompilerParams()
if "needs_layout_passes" in pltpu.CompilerParams.__dataclass_fields__:
    _sc_params = dataclasses.replace(_sc_params, needs_layout_passes=False)


def _fold_body(wz, lz, wh, lh, bz, lzb, bh, lhb, wf, cf):
    lzt = lz[0:F_OUT, :]
    lht = lh[0:F_OUT, :]
    wf[...] = jnp.concatenate(
        [jnp.dot(wz[...], lzt, preferred_element_type=jnp.float32),
         jnp.dot(wh[...], lht, preferred_element_type=jnp.float32)], axis=1)
    cz = jnp.dot(bz[...], lzt, preferred_element_type=jnp.float32) + lzb[...]
    ch = jnp.dot(bh[...], lht, preferred_element_type=jnp.float32) + lhb[...]
    cf[...] = jnp.concatenate([cz, ch], axis=1)


_fold = pl.pallas_call(
    _fold_body,
    out_shape=[jax.ShapeDtypeStruct((F_IN, FC), jnp.float32),
               jax.ShapeDtypeStruct((1, FC), jnp.float32)],
)


@functools.partial(
    pl.kernel,
    out_type=jax.ShapeDtypeStruct((NC, NP), jnp.float32),
    mesh=_mesh,
    compiler_params=_sc_params,
    scratch_types=[
        pltpu.VMEM_SHARED((NS, NP), jnp.float32),
        pltpu.VMEM((NP,), jnp.float32),
        pltpu.VMEM((EPT_DEG,), jnp.int32),
        pltpu.VMEM((EPT_DEG,), jnp.float32),
        pltpu.VMEM((NS, RPT), jnp.float32),
        pltpu.VMEM((RPT,), jnp.float32),
    ])
def _deg_kernel(col_hbm, ew_hbm, deg_hbm, shp, degloc, colv, ewv, tbuf, sumb):
    c = lax.axis_index("c")
    s = lax.axis_index("s")
    pltpu.sync_copy(col_hbm.at[c, s], colv)
    pltpu.sync_copy(ew_hbm.at[c, s], ewv)

    zero16 = jnp.zeros((16,), jnp.float32)

    @pl.loop(0, NP // 16)
    def _z(i):
        degloc[pl.ds(i * 16, 16)] = zero16

    @pl.loop(0, EPT_DEG // 16)
    def _h(k):
        idx = colv[pl.ds(k * 16, 16)]
        w = ewv[pl.ds(k * 16, 16)]
        plsc.addupdate_scatter(degloc, [idx], w)

    pltpu.sync_copy(degloc, shp.at[s])
    plsc.subcore_barrier()
    for k in range(NS):
        pltpu.sync_copy(shp.at[k, pl.ds(s * RPT, RPT)], tbuf.at[k])

    @pl.loop(0, RPT // 16)
    def _r(j):
        acc = tbuf[0, pl.ds(j * 16, 16)]
        for k in range(1, NS):
            acc = acc + tbuf[k, pl.ds(j * 16, 16)]
        sumb[pl.ds(j * 16, 16)] = acc

    pltpu.sync_copy(sumb, deg_hbm.at[c, pl.ds(s * RPT, RPT)])


def _mm_body(xa_ref, xb_ref, wf_ref, deg_ref, y_ref):
    degs = deg_ref[0] + deg_ref[1]
    dinv = lax.rsqrt(1.0 + degs)[:, None]
    ya = jnp.dot(xa_ref[...], wf_ref[...], preferred_element_type=jnp.float32)
    yb = jnp.dot(xb_ref[...], wf_ref[...], preferred_element_type=jnp.float32)
    y_ref[...] = jnp.concatenate([ya * dinv, yb * dinv], axis=1)[None]


def _matmul(xr, wf, degp):
    return pl.pallas_call(
        _mm_body,
        grid=(NPAIR, _NB),
        in_specs=[
            pl.BlockSpec((_MM_BLK, F_IN), lambda p, i: (2 * p * _NB + i, 0)),
            pl.BlockSpec((_MM_BLK, F_IN),
                         lambda p, i: ((2 * p + 1) * _NB + i, 0)),
            pl.BlockSpec((F_IN, FC), lambda p, i: (0, 0)),
            pl.BlockSpec((NC, _MM_BLK), lambda p, i: (0, i)),
        ],
        out_specs=pl.BlockSpec((1, _MM_BLK, FP), lambda p, i: (p, i, 0)),
        out_shape=jax.ShapeDtypeStruct((NPAIR, NP, FP), jnp.float32),
    )(xr, xr, wf, degp)


@functools.partial(
    pl.kernel,
    out_type=jax.ShapeDtypeStruct((NPAIR * NP, FP), jnp.float32),
    mesh=_mesh,
    compiler_params=_sc_params,
    scratch_types=[
        pltpu.VMEM_SHARED((NP, FP), jnp.float32),
        pltpu.VMEM((NCH_AGG, CH), jnp.float32),
        pltpu.VMEM((CH,), jnp.int32),
        pltpu.VMEM((CH,), jnp.int32),
        pltpu.VMEM((CH,), jnp.int32),
        pltpu.VMEM((CH,), jnp.int32),
        pltpu.VMEM((2, CH), jnp.int32),
        pltpu.VMEM((2, CH), jnp.int32),
        pltpu.VMEM((CH, FP), jnp.float32),
        pltpu.VMEM((CH, FP), jnp.float32),
        pltpu.SemaphoreType.DMA,
        pltpu.SemaphoreType.DMA,
        pltpu.SemaphoreType.DMA,
        pltpu.SemaphoreType.DMA,
        pltpu.SemaphoreType.DMA,
        pltpu.SemaphoreType.DMA,
    ])
def _agg_kernel(y_hbm, rc_hbm, ew_hbm, s_hbm,
                acc, ewv, ridx0, ridx1, cidx0, cidx1, rc0, rc1,
                gbuf0, gbuf1, gsem0, gsem1, ssem0, ssem1, isem0, isem1):
    c = lax.axis_index("c")
    s = lax.axis_index("s")
    pltpu.sync_copy(ew_hbm.at[s], ewv)
    rbase = s * RPT

    def _scale(gbuf, i):
        @pl.loop(0, CH // 16)
        def _grp(j):
            w16 = ewv[i, pl.ds(j * 16, 16)]
            base = j * 16
            for l in range(16):
                w = jnp.broadcast_to(w16[l], (16,))
                for f in range(FP // 16):
                    fsl = (base + l, pl.ds(f * 16, 16))
                    gbuf[fsl] = gbuf[fsl] * w

    for q in range(PPC):
        p = c * PPC + q
        pbase = p * NP
        pltpu.sync_copy(y_hbm.at[pl.ds(pbase + rbase, RPT)],
                        acc.at[pl.ds(rbase, RPT)])
        plsc.subcore_barrier()

        bufs = ((ridx0, cidx0, rc0, gbuf0, gsem0, ssem0, isem0),
                (ridx1, cidx1, rc1, gbuf1, gsem1, ssem1, isem1))

        for b, (ridx, cidx, rc, gbuf, gsem, ssem, isem) in enumerate(bufs):
            pltpu.async_copy(rc_hbm.at[s, b], rc, isem)

        @pl.loop(0, NCH_AGG // 2)
        def _chunk(k):
            for b, (ridx, cidx, rc, gbuf, gsem, ssem, isem) in enumerate(bufs):
                i = 2 * k + b

                @pl.when(k > 0)
                def _drain():
                    pltpu.make_async_copy(gbuf, acc.at[cidx], ssem).wait()

                pltpu.make_async_copy(rc_hbm.at[s, i], rc, isem).wait()
                for j in range(CH // 16):
                    sl = pl.ds(j * 16, 16)
                    ridx[sl] = rc[0, sl] + pbase
                    cidx[sl] = rc[1, sl]

                @pl.when(i + 2 < NCH_AGG)
                def _prefetch():
                    pltpu.async_copy(rc_hbm.at[s, i + 2], rc, isem)

                pltpu.async_copy(y_hbm.at[ridx], gbuf, gsem)

            for b, (ridx, cidx, rc, gbuf, gsem, ssem, isem) in enumerate(bufs):
                i = 2 * k + b
                pltpu.make_async_copy(y_hbm.at[ridx], gbuf, gsem).wait()
                _scale(gbuf, i)
                pltpu.async_copy(gbuf, acc.at[cidx], ssem, add=True)

        for ridx, cidx, rc, gbuf, gsem, ssem, isem in bufs:
            pltpu.make_async_copy(gbuf, acc.at[cidx], ssem).wait()
        plsc.subcore_barrier()
        pltpu.sync_copy(acc.at[pl.ds(rbase, RPT)],
                        s_hbm.at[pl.ds(pbase + rbase, RPT)])
        plsc.subcore_barrier()


def _post_body(acc_ref, deg_ref, cf_ref, ow_ref, ob_ref, o_ref):
    degs = deg_ref[0] + deg_ref[1]
    dinv = lax.rsqrt(1.0 + degs)[:, None]
    v = acc_ref[...] * dinv
    outs = []
    for q in range(2):
        z = jax.nn.sigmoid(v[:, q * FC:q * FC + F_OUT]
                           + cf_ref[0, 0:F_OUT])
        ht = jnp.tanh(v[:, q * FC + F_OUT:(q + 1) * FC]
                      + cf_ref[0, F_OUT:FC])
        h = (1.0 - z) * ht
        outs.append(jnp.sum(h * ow_ref[0, :], axis=1, keepdims=True))
    o_ref[...] = jax.nn.sigmoid(jnp.concatenate(outs, axis=1) + ob_ref[...])


def _post(accr, degp, cf, owr, ob):
    return pl.pallas_call(
        _post_body,
        grid=(NPAIR * _NB,),
        in_specs=[
            pl.BlockSpec((_MM_BLK, FP), lambda i: (i, 0)),
            pl.BlockSpec((NC, _MM_BLK), lambda i: (0, i % _NB)),
            pl.BlockSpec((1, FC), lambda i: (0, 0)),
            pl.BlockSpec((1, F_OUT), lambda i: (0, 0)),
            pl.BlockSpec((1, 1), lambda i: (0, 0)),
        ],
        out_specs=pl.BlockSpec((_MM_BLK, 2), lambda i: (i, 0)),
        out_shape=jax.ShapeDtypeStruct((NPAIR * NP, 2), jnp.float32),
    )(accr, degp, cf, owr, ob)


def kernel(x, edge_index, edge_weight, Wz, bz, Wr, br, Wh, bh,
           Lz_w, Lz_b, Lr_w, Lr_b, Lh_w, Lh_b, out_w, out_b):
    ei = edge_index.astype(jnp.int32)
    pad = E_PAD - E
    row = jnp.concatenate([ei[0], jnp.zeros((pad,), jnp.int32)])
    col = jnp.concatenate([ei[1], jnp.zeros((pad,), jnp.int32)])
    ew = jnp.concatenate([edge_weight.astype(jnp.float32),
                          jnp.zeros((pad,), jnp.float32)])

    wf, cf = _fold(Wz, Lz_w, Wh, Lh_w,
                   bz.reshape(1, F_OUT), Lz_b.reshape(1, F_OUT),
                   bh.reshape(1, F_OUT), Lh_b.reshape(1, F_OUT))
    degp = _deg_kernel(col.reshape(NC, NS, EPT_DEG),
                       ew.reshape(NC, NS, EPT_DEG))
    xp = jnp.concatenate(
        [x, jnp.zeros((B, NP - N, F_IN), jnp.float32)], axis=1)
    y2 = _matmul(xp.reshape(B * NP, F_IN), wf, degp)
    rc = jnp.stack([row.reshape(NS, NCH_AGG, CH),
                    col.reshape(NS, NCH_AGG, CH)], axis=2)
    accs = _agg_kernel(y2.reshape(NPAIR * NP, FP), rc,
                       ew.reshape(NS, NCH_AGG, CH))
    o = _post(accs, degp, cf, out_w.reshape(1, F_OUT), out_b.reshape(1, 1))
    return o.reshape(NPAIR, NP, 2).transpose(0, 2, 1).reshape(B, NP, 1)[:, :N]
